# baseline (speedup 1.0000x reference)
import functools

import jax
import jax.numpy as jnp
from jax import lax
from jax.experimental import pallas as pl
from jax.experimental.pallas import tpu as pltpu

USER = 25000
ITEM = 25000
N = USER + ITEM
E = 800000
LATDIM = 64
RIS_ADJ_LAMBDA = 0.2
RIS_LAMBDA = 0.5

_FEAT_BLK = 1000


def _feats_body(x_ref, w_ref, b_ref, o_ref):
    acc = jnp.dot(x_ref[...], w_ref[...], preferred_element_type=jnp.float32)
    acc = acc + b_ref[...]
    n = jnp.sqrt(jnp.sum(acc * acc, axis=1, keepdims=True))
    o_ref[...] = acc / jnp.maximum(n, 1e-12)


def _proj_l2(x, w, b):
    m, k = x.shape
    d = w.shape[1]
    grid = m // _FEAT_BLK
    return pl.pallas_call(
        _feats_body,
        grid=(grid,),
        in_specs=[
            pl.BlockSpec((_FEAT_BLK, k), lambda i: (i, 0)),
            pl.BlockSpec((k, d), lambda i: (0, 0)),
            pl.BlockSpec((1, d), lambda i: (0, 0)),
        ],
        out_specs=pl.BlockSpec((_FEAT_BLK, d), lambda i: (i, 0)),
        out_shape=jax.ShapeDtypeStruct((m, d), jnp.float32),
    )(x, w, b.reshape(1, d))


def _spmm(idx, val, x):
    gathered = val[:, None] * jnp.take(x, idx[1], axis=0)
    return jax.ops.segment_sum(gathered, idx[0], num_segments=N)


_FUSE_BLK = 1000


def _modal_body(ia, ib, iadj, ta, tb, tadj, wi, wt, o_ref):
    img = ia[...] + ib[...] + RIS_ADJ_LAMBDA * iadj[...]
    txt = ta[...] + tb[...] + RIS_ADJ_LAMBDA * tadj[...]
    o_ref[...] = wi[...] * img + wt[...] * txt


def _modal_combine(im1, im2, imadj, tx1, tx2, txadj, wi, wt):
    spec = pl.BlockSpec((_FUSE_BLK, LATDIM), lambda i: (i, 0))
    wspec = pl.BlockSpec((_FUSE_BLK, 1), lambda i: (i, 0))
    return pl.pallas_call(
        _modal_body,
        grid=(N // _FUSE_BLK,),
        in_specs=[spec, spec, spec, spec, spec, spec, wspec, wspec],
        out_specs=spec,
        out_shape=jax.ShapeDtypeStruct((N, LATDIM), jnp.float32),
    )(im1, im2, imadj, tx1, tx2, txadj, wi, wt)


def _final_body(m, g1, g2, o_ref):
    modal = m[...]
    n = jnp.sqrt(jnp.sum(modal * modal, axis=1, keepdims=True))
    o_ref[...] = modal + g1[...] + g2[...] + RIS_LAMBDA * (
        modal / jnp.maximum(n, 1e-12))


def _final_combine(modal, g1, g2):
    spec = pl.BlockSpec((_FUSE_BLK, LATDIM), lambda i: (i, 0))
    return pl.pallas_call(
        _final_body,
        grid=(N // _FUSE_BLK,),
        in_specs=[spec, spec, spec],
        out_specs=spec,
        out_shape=jax.ShapeDtypeStruct((N, LATDIM), jnp.float32),
    )(modal, g1, g2)


def kernel(adj_idx, adj_val, image_adj_idx, image_adj_val, text_adj_idx,
           text_adj_val, att_image_list, att_text_list, uEmbeds, iEmbeds,
           image_embedding, text_embedding, Wi, bi, Wt, bt):
    image_feats_n = _proj_l2(image_embedding, Wi, bi)
    text_feats_n = _proj_l2(text_embedding, Wt, bt)

    ui = jnp.concatenate([uEmbeds, iEmbeds], axis=0)
    embedsImageAdj = _spmm(image_adj_idx, image_adj_val, ui)
    embedsTextAdj = _spmm(text_adj_idx, text_adj_val, ui)

    embedsImage1 = _spmm(adj_idx, adj_val,
                         jnp.concatenate([uEmbeds, image_feats_n], axis=0))
    embedsImage2 = _spmm(adj_idx, adj_val,
                         jnp.concatenate([embedsImage1[:USER], iEmbeds], axis=0))
    embedsText1 = _spmm(adj_idx, adj_val,
                        jnp.concatenate([uEmbeds, text_feats_n], axis=0))
    embedsText2 = _spmm(adj_idx, adj_val,
                        jnp.concatenate([embedsText1[:USER], iEmbeds], axis=0))

    weight_sum = att_image_list + att_text_list
    weight_sum = jnp.where(weight_sum == 0, jnp.ones_like(weight_sum), weight_sum)
    wi_att = (att_image_list / weight_sum)[:, None]
    wt_att = (att_text_list / weight_sum)[:, None]

    embedsModal = _modal_combine(embedsImage1, embedsImage2, embedsImageAdj,
                                 embedsText1, embedsText2, embedsTextAdj,
                                 wi_att, wt_att)

    g1 = _spmm(adj_idx, adj_val, embedsModal)
    g2 = _spmm(adj_idx, adj_val, g1)
    embeds = _final_combine(embedsModal, g1, g2)
    return (embeds[:USER], embeds[USER:])


# speedup vs baseline: 2.9807x; 2.9807x over previous
import functools

import jax
import jax.numpy as jnp
from jax import lax
from jax.experimental import pallas as pl
from jax.experimental.pallas import tpu as pltpu
from jax.experimental.pallas import tpu_sc as plsc

USER = 25000
ITEM = 25000
N = USER + ITEM
E = 800000
LATDIM = 64
RIS_ADJ_LAMBDA = 0.2
RIS_LAMBDA = 0.5

H = N // 2
ACC_ROWS = 25088
TRASH = H
NSUB = 16
SPAN = E // NSUB
MEGA = 2000
CHUNK = 80
NM = SPAN // MEGA
NK = MEGA // CHUNK
ZROWS = 1568


def _spmm_body(rows_hbm, cols_hbm, vals_hbm, x_hbm, zeros_hbm, out_hbm,
               acc, rows_v, cols_v, vals_v, lidx_v, gath_v, sem):
    core = lax.axis_index("c")
    sid = lax.axis_index("s")
    coff = core * H

    pltpu.sync_copy(zeros_hbm.at[pl.ds(0, ACC_ROWS // NSUB)],
                    acc.at[pl.ds(sid * (ACC_ROWS // NSUB), ACC_ROWS // NSUB)])
    plsc.subcore_barrier()

    def mega_body(m, _):
        base = sid * SPAN + m * MEGA
        pltpu.sync_copy(rows_hbm.at[pl.ds(base, MEGA)], rows_v)
        pltpu.sync_copy(cols_hbm.at[pl.ds(base // CHUNK, NK)], cols_v)
        pltpu.sync_copy(vals_hbm.at[pl.ds(base, MEGA)], vals_v)

        def chunk_body(k, _):
            off = k * CHUNK
            for j in range(CHUNK // 16):
                r16 = rows_v[pl.ds(off + j * 16, 16)]
                lr = r16 - coff
                ok = (lr >= 0) & (lr < H)
                lidx_v[pl.ds(j * 16, 16)] = jnp.where(ok, lr, TRASH)
            pltpu.async_copy(x_hbm.at[cols_v.at[k]],
                             gath_v, sem).wait()
            for e in range(CHUNK):
                vidx = jnp.broadcast_to(off + e, (16,)).astype(jnp.int32)
                vv = plsc.load_gather(vals_v, [vidx])
                for c in range(LATDIM // 16):
                    sl = pl.ds(c * 16, 16)
                    gath_v[e, sl] = gath_v[e, sl] * vv
            pltpu.sync_copy(gath_v, acc.at[lidx_v], add=True)
            return 0

        lax.fori_loop(0, NK, chunk_body, 0)
        return 0

    lax.fori_loop(0, NM, mega_body, 0)
    plsc.subcore_barrier()

    fl = 1560

    @pl.when(sid < 15)
    def _():
        pltpu.sync_copy(acc.at[pl.ds(sid * fl, fl)],
                        out_hbm.at[pl.ds(coff + sid * fl, fl)])

    @pl.when(sid == 15)
    def _():
        pltpu.sync_copy(acc.at[pl.ds(15 * fl, H - 15 * fl)],
                        out_hbm.at[pl.ds(coff + 15 * fl, H - 15 * fl)])


_spmm_call = pl.kernel(
    _spmm_body,
    out_type=jax.ShapeDtypeStruct((N, LATDIM), jnp.float32),
    mesh=plsc.VectorSubcoreMesh(core_axis_name="c", subcore_axis_name="s"),
    compiler_params=pltpu.CompilerParams(
        needs_layout_passes=False, use_tc_tiling_on_sc=False),
    scratch_types=[
        pltpu.VMEM_SHARED((ACC_ROWS, LATDIM), jnp.float32),
        pltpu.VMEM((MEGA,), jnp.int32),
        pltpu.VMEM((NK, CHUNK), jnp.int32),
        pltpu.VMEM((MEGA,), jnp.float32),
        pltpu.VMEM((CHUNK,), jnp.int32),
        pltpu.VMEM((CHUNK, LATDIM), jnp.float32),
        pltpu.SemaphoreType.DMA,
    ],
)

_ZEROS = None


def _spmm(idx, val, x):
    global _ZEROS
    if _ZEROS is None:
        _ZEROS = jnp.zeros((ZROWS, LATDIM), jnp.float32)
    return _spmm_call(idx[0], idx[1].reshape(E // CHUNK, CHUNK), val, x, _ZEROS)


_FEAT_BLK = 1000


def _feats_body(x_ref, w_ref, b_ref, o_ref):
    acc = jnp.dot(x_ref[...], w_ref[...], preferred_element_type=jnp.float32)
    acc = acc + b_ref[...]
    n = jnp.sqrt(jnp.sum(acc * acc, axis=1, keepdims=True))
    o_ref[...] = acc / jnp.maximum(n, 1e-12)


def _proj_l2(x, w, b):
    m, k = x.shape
    d = w.shape[1]
    grid = m // _FEAT_BLK
    return pl.pallas_call(
        _feats_body,
        grid=(grid,),
        in_specs=[
            pl.BlockSpec((_FEAT_BLK, k), lambda i: (i, 0)),
            pl.BlockSpec((k, d), lambda i: (0, 0)),
            pl.BlockSpec((1, d), lambda i: (0, 0)),
        ],
        out_specs=pl.BlockSpec((_FEAT_BLK, d), lambda i: (i, 0)),
        out_shape=jax.ShapeDtypeStruct((m, d), jnp.float32),
    )(x, w, b.reshape(1, d))


_FUSE_BLK = 1000


def _modal_body(ia, ib, iadj, ta, tb, tadj, wi, wt, o_ref):
    img = ia[...] + ib[...] + RIS_ADJ_LAMBDA * iadj[...]
    txt = ta[...] + tb[...] + RIS_ADJ_LAMBDA * tadj[...]
    o_ref[...] = wi[...] * img + wt[...] * txt


def _modal_combine(im1, im2, imadj, tx1, tx2, txadj, wi, wt):
    spec = pl.BlockSpec((_FUSE_BLK, LATDIM), lambda i: (i, 0))
    wspec = pl.BlockSpec((_FUSE_BLK, 1), lambda i: (i, 0))
    return pl.pallas_call(
        _modal_body,
        grid=(N // _FUSE_BLK,),
        in_specs=[spec, spec, spec, spec, spec, spec, wspec, wspec],
        out_specs=spec,
        out_shape=jax.ShapeDtypeStruct((N, LATDIM), jnp.float32),
    )(im1, im2, imadj, tx1, tx2, txadj, wi, wt)


def _final_body(m, g1, g2, o_ref):
    modal = m[...]
    n = jnp.sqrt(jnp.sum(modal * modal, axis=1, keepdims=True))
    o_ref[...] = modal + g1[...] + g2[...] + RIS_LAMBDA * (
        modal / jnp.maximum(n, 1e-12))


def _final_combine(modal, g1, g2):
    spec = pl.BlockSpec((_FUSE_BLK, LATDIM), lambda i: (i, 0))
    return pl.pallas_call(
        _final_body,
        grid=(N // _FUSE_BLK,),
        in_specs=[spec, spec, spec],
        out_specs=spec,
        out_shape=jax.ShapeDtypeStruct((N, LATDIM), jnp.float32),
    )(modal, g1, g2)


def kernel(adj_idx, adj_val, image_adj_idx, image_adj_val, text_adj_idx,
           text_adj_val, att_image_list, att_text_list, uEmbeds, iEmbeds,
           image_embedding, text_embedding, Wi, bi, Wt, bt):
    image_feats_n = _proj_l2(image_embedding, Wi, bi)
    text_feats_n = _proj_l2(text_embedding, Wt, bt)

    ui = jnp.concatenate([uEmbeds, iEmbeds], axis=0)
    embedsImageAdj = _spmm(image_adj_idx, image_adj_val, ui)
    embedsTextAdj = _spmm(text_adj_idx, text_adj_val, ui)

    embedsImage1 = _spmm(adj_idx, adj_val,
                         jnp.concatenate([uEmbeds, image_feats_n], axis=0))
    embedsImage2 = _spmm(adj_idx, adj_val,
                         jnp.concatenate([embedsImage1[:USER], iEmbeds], axis=0))
    embedsText1 = _spmm(adj_idx, adj_val,
                        jnp.concatenate([uEmbeds, text_feats_n], axis=0))
    embedsText2 = _spmm(adj_idx, adj_val,
                        jnp.concatenate([embedsText1[:USER], iEmbeds], axis=0))

    weight_sum = att_image_list + att_text_list
    weight_sum = jnp.where(weight_sum == 0, jnp.ones_like(weight_sum), weight_sum)
    wi_att = (att_image_list / weight_sum)[:, None]
    wt_att = (att_text_list / weight_sum)[:, None]

    embedsModal = _modal_combine(embedsImage1, embedsImage2, embedsImageAdj,
                                 embedsText1, embedsText2, embedsTextAdj,
                                 wi_att, wt_att)

    g1 = _spmm(adj_idx, adj_val, embedsModal)
    g2 = _spmm(adj_idx, adj_val, g1)
    embeds = _final_combine(embedsModal, g1, g2)
    return (embeds[:USER], embeds[USER:])


# speedup vs baseline: 4.1687x; 1.3986x over previous
import jax
import jax.numpy as jnp
from jax import lax
from jax.experimental import pallas as pl
from jax.experimental.pallas import tpu as pltpu
from jax.experimental.pallas import tpu_sc as plsc

USER = 25000
ITEM = 25000
N = USER + ITEM
E = 800000
LATDIM = 64
HD = LATDIM // 2
RIS_ADJ_LAMBDA = 0.2
RIS_LAMBDA = 0.5

NSUB = 16
SPAN = E // NSUB
CHUNK = 80
MEGA = 2000
NK = MEGA // CHUNK
NM = SPAN // MEGA
ACC_ROWS = 50048
ZROWS = ACC_ROWS // NSUB


def _spmm_body(rows_hbm, cols_hbm, vals_hbm, xs_hbm, zeros_hbm, out_hbm,
               acc, cols_v, vals_v, rows_a, rows_b, gath_a, gath_b,
               sem_s, sem_a, sem_b):
    core = lax.axis_index("c")
    sid = lax.axis_index("s")

    pltpu.sync_copy(zeros_hbm, acc.at[pl.ds(sid * ZROWS, ZROWS)])
    plsc.subcore_barrier()

    xsrc = xs_hbm.at[core]

    def start(cb, k, rows_buf, gath_buf, sem):
        pltpu.async_copy(rows_hbm.at[cb + k], rows_buf, sem)
        pltpu.async_copy(xsrc.at[cols_v.at[k]], gath_buf, sem)

    def wait(cb, k, rows_buf, gath_buf, sem):
        pltpu.make_async_copy(rows_hbm.at[cb + k], rows_buf, sem).wait()
        pltpu.make_async_copy(xsrc.at[cols_v.at[k]], gath_buf, sem).wait()

    def finish(k, rows_buf, gath_buf):
        off = k * CHUNK
        for e in range(CHUNK):
            vidx = jnp.broadcast_to(off + e, (16,)).astype(jnp.int32)
            vv = plsc.load_gather(vals_v, [vidx])
            for c in range(HD // 16):
                sl = pl.ds(c * 16, 16)
                gath_buf[e, sl] = gath_buf[e, sl] * vv
        pltpu.sync_copy(gath_buf, acc.at[rows_buf], add=True)

    def mega_body(m, _):
        base = sid * SPAN + m * MEGA
        cb = base // CHUNK
        d_cols = pltpu.async_copy(cols_hbm.at[pl.ds(cb, NK)], cols_v, sem_s)
        d_vals = pltpu.async_copy(vals_hbm.at[pl.ds(base, MEGA)], vals_v,
                                  sem_s)
        d_cols.wait()
        d_vals.wait()
        start(cb, 0, rows_a, gath_a, sem_a)

        def pair_body(t, _):
            k0 = 2 * t
            wait(cb, k0, rows_a, gath_a, sem_a)
            start(cb, k0 + 1, rows_b, gath_b, sem_b)
            finish(k0, rows_a, gath_a)
            wait(cb, k0 + 1, rows_b, gath_b, sem_b)
            start(cb, k0 + 2, rows_a, gath_a, sem_a)
            finish(k0 + 1, rows_b, gath_b)
            return 0

        lax.fori_loop(0, (NK - 1) // 2, pair_body, 0)
        wait(cb, NK - 1, rows_a, gath_a, sem_a)
        finish(NK - 1, rows_a, gath_a)
        return 0

    lax.fori_loop(0, NM, mega_body, 0)
    plsc.subcore_barrier()

    fl = 3120

    @pl.when(sid < 15)
    def _():
        pltpu.sync_copy(acc.at[pl.ds(sid * fl, fl)],
                        out_hbm.at[core, pl.ds(sid * fl, fl)])

    @pl.when(sid == 15)
    def _():
        pltpu.sync_copy(acc.at[pl.ds(15 * fl, N - 15 * fl)],
                        out_hbm.at[core, pl.ds(15 * fl, N - 15 * fl)])


_spmm_call = pl.kernel(
    _spmm_body,
    out_type=jax.ShapeDtypeStruct((2, N, HD), jnp.float32),
    mesh=plsc.VectorSubcoreMesh(core_axis_name="c", subcore_axis_name="s"),
    compiler_params=pltpu.CompilerParams(
        needs_layout_passes=False, use_tc_tiling_on_sc=False),
    scratch_types=[
        pltpu.VMEM_SHARED((ACC_ROWS, HD), jnp.float32),
        pltpu.VMEM((NK, CHUNK), jnp.int32),
        pltpu.VMEM((MEGA,), jnp.float32),
        pltpu.VMEM((CHUNK,), jnp.int32),
        pltpu.VMEM((CHUNK,), jnp.int32),
        pltpu.VMEM((CHUNK, HD), jnp.float32),
        pltpu.VMEM((CHUNK, HD), jnp.float32),
        pltpu.SemaphoreType.DMA,
        pltpu.SemaphoreType.DMA,
        pltpu.SemaphoreType.DMA,
    ],
)

_ZEROS = None


def _spmm(idx, val, xs):
    global _ZEROS
    if _ZEROS is None:
        _ZEROS = jnp.zeros((ZROWS, HD), jnp.float32)
    rows2 = idx[0].reshape(E // CHUNK, CHUNK)
    cols2 = idx[1].reshape(E // CHUNK, CHUNK)
    return _spmm_call(rows2, cols2, val, xs, _ZEROS)


def _split(x):
    return jnp.stack([x[:, :HD], x[:, HD:]])


_FEAT_BLK = 1000


def _feats_body(x_ref, w_ref, b_ref, o_ref):
    acc = jnp.dot(x_ref[...], w_ref[...], preferred_element_type=jnp.float32)
    acc = acc + b_ref[...]
    n = jnp.sqrt(jnp.sum(acc * acc, axis=1, keepdims=True))
    o_ref[...] = acc / jnp.maximum(n, 1e-12)


def _proj_l2(x, w, b):
    m, k = x.shape
    d = w.shape[1]
    grid = m // _FEAT_BLK
    return pl.pallas_call(
        _feats_body,
        grid=(grid,),
        in_specs=[
            pl.BlockSpec((_FEAT_BLK, k), lambda i: (i, 0)),
            pl.BlockSpec((k, d), lambda i: (0, 0)),
            pl.BlockSpec((1, d), lambda i: (0, 0)),
        ],
        out_specs=pl.BlockSpec((_FEAT_BLK, d), lambda i: (i, 0)),
        out_shape=jax.ShapeDtypeStruct((m, d), jnp.float32),
    )(x, w, b.reshape(1, d))


_FUSE_BLK = 1000


def _modal_body(ia, ib, iadj, ta, tb, tadj, wi, wt, o_ref):
    img = ia[...] + ib[...] + RIS_ADJ_LAMBDA * iadj[...]
    txt = ta[...] + tb[...] + RIS_ADJ_LAMBDA * tadj[...]
    o_ref[...] = wi[...] * img + wt[...] * txt


def _modal_combine(im1, im2, imadj, tx1, tx2, txadj, wi, wt):
    spec = pl.BlockSpec((2, _FUSE_BLK, HD), lambda i: (0, i, 0))
    wspec = pl.BlockSpec((_FUSE_BLK, 1), lambda i: (i, 0))
    return pl.pallas_call(
        _modal_body,
        grid=(N // _FUSE_BLK,),
        in_specs=[spec, spec, spec, spec, spec, spec, wspec, wspec],
        out_specs=spec,
        out_shape=jax.ShapeDtypeStruct((2, N, HD), jnp.float32),
    )(im1, im2, imadj, tx1, tx2, txadj, wi, wt)


def _final_body(m, g1, g2, o_ref):
    m0, m1 = m[0], m[1]
    s = jnp.sum(m0 * m0 + m1 * m1, axis=1, keepdims=True)
    inv = RIS_LAMBDA / jnp.maximum(jnp.sqrt(s), 1e-12)
    o_ref[:, :HD] = m0 + g1[0] + g2[0] + inv * m0
    o_ref[:, HD:] = m1 + g1[1] + g2[1] + inv * m1


def _final_combine(modal, g1, g2):
    spec = pl.BlockSpec((2, _FUSE_BLK, HD), lambda i: (0, i, 0))
    return pl.pallas_call(
        _final_body,
        grid=(N // _FUSE_BLK,),
        in_specs=[spec, spec, spec],
        out_specs=pl.BlockSpec((_FUSE_BLK, LATDIM), lambda i: (i, 0)),
        out_shape=jax.ShapeDtypeStruct((N, LATDIM), jnp.float32),
    )(modal, g1, g2)


def kernel(adj_idx, adj_val, image_adj_idx, image_adj_val, text_adj_idx,
           text_adj_val, att_image_list, att_text_list, uEmbeds, iEmbeds,
           image_embedding, text_embedding, Wi, bi, Wt, bt):
    image_feats_n = _proj_l2(image_embedding, Wi, bi)
    text_feats_n = _proj_l2(text_embedding, Wt, bt)

    u_s = _split(uEmbeds)
    i_s = _split(iEmbeds)
    ui_s = jnp.concatenate([u_s, i_s], axis=1)

    embedsImageAdj = _spmm(image_adj_idx, image_adj_val, ui_s)
    embedsTextAdj = _spmm(text_adj_idx, text_adj_val, ui_s)

    embedsImage1 = _spmm(adj_idx, adj_val,
                         jnp.concatenate([u_s, _split(image_feats_n)], axis=1))
    embedsImage2 = _spmm(adj_idx, adj_val,
                         jnp.concatenate([embedsImage1[:, :USER], i_s], axis=1))
    embedsText1 = _spmm(adj_idx, adj_val,
                        jnp.concatenate([u_s, _split(text_feats_n)], axis=1))
    embedsText2 = _spmm(adj_idx, adj_val,
                        jnp.concatenate([embedsText1[:, :USER], i_s], axis=1))

    weight_sum = att_image_list + att_text_list
    weight_sum = jnp.where(weight_sum == 0, jnp.ones_like(weight_sum), weight_sum)
    wi_att = (att_image_list / weight_sum)[:, None]
    wt_att = (att_text_list / weight_sum)[:, None]

    embedsModal = _modal_combine(embedsImage1, embedsImage2, embedsImageAdj,
                                 embedsText1, embedsText2, embedsTextAdj,
                                 wi_att, wt_att)

    g1 = _spmm(adj_idx, adj_val, embedsModal)
    g2 = _spmm(adj_idx, adj_val, g1)
    embeds = _final_combine(embedsModal, g1, g2)
    return (embeds[:USER], embeds[USER:])


# speedup vs baseline: 5.5972x; 1.3427x over previous
import jax
import jax.numpy as jnp
from jax import lax
from jax.experimental import pallas as pl
from jax.experimental.pallas import tpu as pltpu
from jax.experimental.pallas import tpu_sc as plsc

USER = 25000
ITEM = 25000
N = USER + ITEM
E = 800000
LATDIM = 64
HD = LATDIM // 2
RIS_ADJ_LAMBDA = 0.2
RIS_LAMBDA = 0.5

NSUB = 16
SPAN = E // NSUB
CHUNK = 80
MEGA = 2000
NK = MEGA // CHUNK
NM = SPAN // MEGA
ACC_ROWS = 50048
ZROWS = ACC_ROWS // NSUB


def _spmm_body(rows_hbm, cols_hbm, vals_hbm, xs_hbm, zeros_hbm, out_hbm,
               acc, cols_v, vals_v, rows_a, rows_b, gath_a, gath_b,
               sem_s, sem_a, sem_b, sem_sa, sem_sb):
    core = lax.axis_index("c")
    sid = lax.axis_index("s")

    pltpu.sync_copy(zeros_hbm, acc.at[pl.ds(sid * ZROWS, ZROWS)])

    xsrc = xs_hbm.at[core]

    def start(cb, k, rows_buf, gath_buf, sem):
        pltpu.async_copy(rows_hbm.at[cb + k], rows_buf, sem)
        pltpu.async_copy(xsrc.at[cols_v.at[k]], gath_buf, sem)

    def wait(cb, k, rows_buf, gath_buf, sem):
        pltpu.make_async_copy(rows_hbm.at[cb + k], rows_buf, sem).wait()
        pltpu.make_async_copy(xsrc.at[cols_v.at[k]], gath_buf, sem).wait()

    def compute(k, gath_buf):
        off = k * CHUNK
        for j in range(CHUNK // 16):
            val16 = vals_v[pl.ds(off + j * 16, 16)]
            for i in range(16):
                e = j * 16 + i
                vv = lax.gather(
                    val16, jnp.full((16, 1), i, jnp.int32),
                    lax.GatherDimensionNumbers(
                        offset_dims=(), collapsed_slice_dims=(0,),
                        start_index_map=(0,)),
                    slice_sizes=(1,),
                    mode=lax.GatherScatterMode.PROMISE_IN_BOUNDS)
                for c in range(HD // 16):
                    sl = pl.ds(c * 16, 16)
                    gath_buf[e, sl] = gath_buf[e, sl] * vv

    def scat_start(rows_buf, gath_buf, ssem):
        pltpu.async_copy(gath_buf, acc.at[rows_buf], ssem, add=True)

    def scat_wait(rows_buf, gath_buf, ssem):
        pltpu.make_async_copy(gath_buf, acc.at[rows_buf], ssem).wait()

    for buf in (gath_a, gath_b):
        for e in range(CHUNK):
            for c in range(HD // 16):
                buf[e, pl.ds(c * 16, 16)] = jnp.zeros((16,), jnp.float32)
    for e in range(CHUNK // 16):
        rows_a[pl.ds(e * 16, 16)] = jnp.zeros((16,), jnp.int32)
        rows_b[pl.ds(e * 16, 16)] = jnp.zeros((16,), jnp.int32)
    plsc.subcore_barrier()
    scat_start(rows_a, gath_a, sem_sa)
    scat_start(rows_b, gath_b, sem_sb)

    def mega_body(m, _):
        base = sid * SPAN + m * MEGA
        cb = base // CHUNK
        d_cols = pltpu.async_copy(cols_hbm.at[pl.ds(cb, NK)], cols_v, sem_s)
        d_vals = pltpu.async_copy(vals_hbm.at[pl.ds(base, MEGA)], vals_v,
                                  sem_s)
        d_cols.wait()
        d_vals.wait()
        scat_wait(rows_a, gath_a, sem_sa)
        start(cb, 0, rows_a, gath_a, sem_a)

        def pair_body(t, _):
            k0 = 2 * t
            wait(cb, k0, rows_a, gath_a, sem_a)
            scat_wait(rows_b, gath_b, sem_sb)
            start(cb, k0 + 1, rows_b, gath_b, sem_b)
            compute(k0, gath_a)
            scat_start(rows_a, gath_a, sem_sa)
            wait(cb, k0 + 1, rows_b, gath_b, sem_b)
            scat_wait(rows_a, gath_a, sem_sa)
            start(cb, k0 + 2, rows_a, gath_a, sem_a)
            compute(k0 + 1, gath_b)
            scat_start(rows_b, gath_b, sem_sb)
            return 0

        lax.fori_loop(0, (NK - 1) // 2, pair_body, 0)
        wait(cb, NK - 1, rows_a, gath_a, sem_a)
        compute(NK - 1, gath_a)
        scat_start(rows_a, gath_a, sem_sa)
        return 0

    lax.fori_loop(0, NM, mega_body, 0)
    scat_wait(rows_a, gath_a, sem_sa)
    scat_wait(rows_b, gath_b, sem_sb)
    plsc.subcore_barrier()

    fl = 3120

    @pl.when(sid < 15)
    def _():
        pltpu.sync_copy(acc.at[pl.ds(sid * fl, fl)],
                        out_hbm.at[core, pl.ds(sid * fl, fl)])

    @pl.when(sid == 15)
    def _():
        pltpu.sync_copy(acc.at[pl.ds(15 * fl, N - 15 * fl)],
                        out_hbm.at[core, pl.ds(15 * fl, N - 15 * fl)])


_spmm_call = pl.kernel(
    _spmm_body,
    out_type=jax.ShapeDtypeStruct((2, N, HD), jnp.float32),
    mesh=plsc.VectorSubcoreMesh(core_axis_name="c", subcore_axis_name="s"),
    compiler_params=pltpu.CompilerParams(
        needs_layout_passes=False, use_tc_tiling_on_sc=False),
    scratch_types=[
        pltpu.VMEM_SHARED((ACC_ROWS, HD), jnp.float32),
        pltpu.VMEM((NK, CHUNK), jnp.int32),
        pltpu.VMEM((MEGA,), jnp.float32),
        pltpu.VMEM((CHUNK,), jnp.int32),
        pltpu.VMEM((CHUNK,), jnp.int32),
        pltpu.VMEM((CHUNK, HD), jnp.float32),
        pltpu.VMEM((CHUNK, HD), jnp.float32),
        pltpu.SemaphoreType.DMA,
        pltpu.SemaphoreType.DMA,
        pltpu.SemaphoreType.DMA,
        pltpu.SemaphoreType.DMA,
        pltpu.SemaphoreType.DMA,
    ],
)

_ZEROS = None


def _spmm(idx, val, xs):
    global _ZEROS
    if _ZEROS is None:
        _ZEROS = jnp.zeros((ZROWS, HD), jnp.float32)
    rows2 = idx[0].reshape(E // CHUNK, CHUNK)
    cols2 = idx[1].reshape(E // CHUNK, CHUNK)
    return _spmm_call(rows2, cols2, val, xs, _ZEROS)


def _split(x):
    return jnp.stack([x[:, :HD], x[:, HD:]])


_FEAT_BLK = 1000


def _feats_body(x_ref, w_ref, b_ref, o_ref):
    acc = jnp.dot(x_ref[...], w_ref[...], preferred_element_type=jnp.float32)
    acc = acc + b_ref[...]
    n = jnp.sqrt(jnp.sum(acc * acc, axis=1, keepdims=True))
    o_ref[...] = acc / jnp.maximum(n, 1e-12)


def _proj_l2(x, w, b):
    m, k = x.shape
    d = w.shape[1]
    grid = m // _FEAT_BLK
    return pl.pallas_call(
        _feats_body,
        grid=(grid,),
        in_specs=[
            pl.BlockSpec((_FEAT_BLK, k), lambda i: (i, 0)),
            pl.BlockSpec((k, d), lambda i: (0, 0)),
            pl.BlockSpec((1, d), lambda i: (0, 0)),
        ],
        out_specs=pl.BlockSpec((_FEAT_BLK, d), lambda i: (i, 0)),
        out_shape=jax.ShapeDtypeStruct((m, d), jnp.float32),
    )(x, w, b.reshape(1, d))


_FUSE_BLK = 1000


def _modal_body(ia, ib, iadj, ta, tb, tadj, wi, wt, o_ref):
    img = ia[...] + ib[...] + RIS_ADJ_LAMBDA * iadj[...]
    txt = ta[...] + tb[...] + RIS_ADJ_LAMBDA * tadj[...]
    o_ref[...] = wi[...] * img + wt[...] * txt


def _modal_combine(im1, im2, imadj, tx1, tx2, txadj, wi, wt):
    spec = pl.BlockSpec((2, _FUSE_BLK, HD), lambda i: (0, i, 0))
    wspec = pl.BlockSpec((_FUSE_BLK, 1), lambda i: (i, 0))
    return pl.pallas_call(
        _modal_body,
        grid=(N // _FUSE_BLK,),
        in_specs=[spec, spec, spec, spec, spec, spec, wspec, wspec],
        out_specs=spec,
        out_shape=jax.ShapeDtypeStruct((2, N, HD), jnp.float32),
    )(im1, im2, imadj, tx1, tx2, txadj, wi, wt)


def _final_body(m, g1, g2, o_ref):
    m0, m1 = m[0], m[1]
    s = jnp.sum(m0 * m0 + m1 * m1, axis=1, keepdims=True)
    inv = RIS_LAMBDA / jnp.maximum(jnp.sqrt(s), 1e-12)
    o_ref[:, :HD] = m0 + g1[0] + g2[0] + inv * m0
    o_ref[:, HD:] = m1 + g1[1] + g2[1] + inv * m1


def _final_combine(modal, g1, g2):
    spec = pl.BlockSpec((2, _FUSE_BLK, HD), lambda i: (0, i, 0))
    return pl.pallas_call(
        _final_body,
        grid=(N // _FUSE_BLK,),
        in_specs=[spec, spec, spec],
        out_specs=pl.BlockSpec((_FUSE_BLK, LATDIM), lambda i: (i, 0)),
        out_shape=jax.ShapeDtypeStruct((N, LATDIM), jnp.float32),
    )(modal, g1, g2)


def kernel(adj_idx, adj_val, image_adj_idx, image_adj_val, text_adj_idx,
           text_adj_val, att_image_list, att_text_list, uEmbeds, iEmbeds,
           image_embedding, text_embedding, Wi, bi, Wt, bt):
    image_feats_n = _proj_l2(image_embedding, Wi, bi)
    text_feats_n = _proj_l2(text_embedding, Wt, bt)

    u_s = _split(uEmbeds)
    i_s = _split(iEmbeds)
    ui_s = jnp.concatenate([u_s, i_s], axis=1)

    embedsImageAdj = _spmm(image_adj_idx, image_adj_val, ui_s)
    embedsTextAdj = _spmm(text_adj_idx, text_adj_val, ui_s)

    embedsImage1 = _spmm(adj_idx, adj_val,
                         jnp.concatenate([u_s, _split(image_feats_n)], axis=1))
    embedsImage2 = _spmm(adj_idx, adj_val,
                         jnp.concatenate([embedsImage1[:, :USER], i_s], axis=1))
    embedsText1 = _spmm(adj_idx, adj_val,
                        jnp.concatenate([u_s, _split(text_feats_n)], axis=1))
    embedsText2 = _spmm(adj_idx, adj_val,
                        jnp.concatenate([embedsText1[:, :USER], i_s], axis=1))

    weight_sum = att_image_list + att_text_list
    weight_sum = jnp.where(weight_sum == 0, jnp.ones_like(weight_sum), weight_sum)
    wi_att = (att_image_list / weight_sum)[:, None]
    wt_att = (att_text_list / weight_sum)[:, None]

    embedsModal = _modal_combine(embedsImage1, embedsImage2, embedsImageAdj,
                                 embedsText1, embedsText2, embedsTextAdj,
                                 wi_att, wt_att)

    g1 = _spmm(adj_idx, adj_val, embedsModal)
    g2 = _spmm(adj_idx, adj_val, g1)
    embeds = _final_combine(embedsModal, g1, g2)
    return (embeds[:USER], embeds[USER:])


# speedup vs baseline: 5.7367x; 1.0249x over previous
import jax
import jax.numpy as jnp
from jax import lax
from jax.experimental import pallas as pl
from jax.experimental.pallas import tpu as pltpu
from jax.experimental.pallas import tpu_sc as plsc

USER = 25000
ITEM = 25000
N = USER + ITEM
E = 800000
LATDIM = 64
HD = LATDIM // 2
RIS_ADJ_LAMBDA = 0.2
RIS_LAMBDA = 0.5

NSUB = 16
SPAN = E // NSUB
CHUNK = 80
MEGA = 10000
NK = MEGA // CHUNK
NM = SPAN // MEGA
ACC_ROWS = 50048
ZROWS = ACC_ROWS // NSUB


def _spmm_body(rows_hbm, cols_hbm, vals_hbm, xs_hbm, zeros_hbm, out_hbm,
               acc, cols_v, vals_v, rows_a, rows_b, gath_a, gath_b,
               sem_s, sem_a, sem_b, sem_sa, sem_sb):
    core = lax.axis_index("c")
    sid = lax.axis_index("s")

    pltpu.sync_copy(zeros_hbm, acc.at[pl.ds(sid * ZROWS, ZROWS)])

    xsrc = xs_hbm.at[core]

    def start(cb, k, rows_buf, gath_buf, sem):
        pltpu.async_copy(rows_hbm.at[cb + k], rows_buf, sem)
        pltpu.async_copy(xsrc.at[cols_v.at[k]], gath_buf, sem)

    def wait(cb, k, rows_buf, gath_buf, sem):
        pltpu.make_async_copy(rows_hbm.at[cb + k], rows_buf, sem).wait()
        pltpu.make_async_copy(xsrc.at[cols_v.at[k]], gath_buf, sem).wait()

    def compute(k, gath_buf):
        off = k * CHUNK
        for j in range(CHUNK // 16):
            val16 = vals_v[pl.ds(off + j * 16, 16)]
            for i in range(16):
                e = j * 16 + i
                vv = lax.gather(
                    val16, jnp.full((16, 1), i, jnp.int32),
                    lax.GatherDimensionNumbers(
                        offset_dims=(), collapsed_slice_dims=(0,),
                        start_index_map=(0,)),
                    slice_sizes=(1,),
                    mode=lax.GatherScatterMode.PROMISE_IN_BOUNDS)
                for c in range(HD // 16):
                    sl = pl.ds(c * 16, 16)
                    gath_buf[e, sl] = gath_buf[e, sl] * vv

    def scat_start(rows_buf, gath_buf, ssem):
        pltpu.async_copy(gath_buf, acc.at[rows_buf], ssem, add=True)

    def scat_wait(rows_buf, gath_buf, ssem):
        pltpu.make_async_copy(gath_buf, acc.at[rows_buf], ssem).wait()

    for buf in (gath_a, gath_b):
        for e in range(CHUNK):
            for c in range(HD // 16):
                buf[e, pl.ds(c * 16, 16)] = jnp.zeros((16,), jnp.float32)
    for e in range(CHUNK // 16):
        rows_a[pl.ds(e * 16, 16)] = jnp.zeros((16,), jnp.int32)
        rows_b[pl.ds(e * 16, 16)] = jnp.zeros((16,), jnp.int32)
    plsc.subcore_barrier()
    scat_start(rows_a, gath_a, sem_sa)
    scat_start(rows_b, gath_b, sem_sb)

    def mega_body(m, _):
        base = sid * SPAN + m * MEGA
        cb = base // CHUNK
        d_cols = pltpu.async_copy(cols_hbm.at[pl.ds(cb, NK)], cols_v, sem_s)
        d_vals = pltpu.async_copy(vals_hbm.at[pl.ds(base, MEGA)], vals_v,
                                  sem_s)
        d_cols.wait()
        d_vals.wait()
        scat_wait(rows_a, gath_a, sem_sa)
        start(cb, 0, rows_a, gath_a, sem_a)

        def pair_body(t, _):
            k0 = 2 * t
            wait(cb, k0, rows_a, gath_a, sem_a)
            scat_wait(rows_b, gath_b, sem_sb)
            start(cb, k0 + 1, rows_b, gath_b, sem_b)
            compute(k0, gath_a)
            scat_start(rows_a, gath_a, sem_sa)
            wait(cb, k0 + 1, rows_b, gath_b, sem_b)
            scat_wait(rows_a, gath_a, sem_sa)
            start(cb, k0 + 2, rows_a, gath_a, sem_a)
            compute(k0 + 1, gath_b)
            scat_start(rows_b, gath_b, sem_sb)
            return 0

        lax.fori_loop(0, (NK - 1) // 2, pair_body, 0)
        wait(cb, NK - 1, rows_a, gath_a, sem_a)
        compute(NK - 1, gath_a)
        scat_start(rows_a, gath_a, sem_sa)
        return 0

    lax.fori_loop(0, NM, mega_body, 0)
    scat_wait(rows_a, gath_a, sem_sa)
    scat_wait(rows_b, gath_b, sem_sb)
    plsc.subcore_barrier()

    fl = 3120

    @pl.when(sid < 15)
    def _():
        pltpu.sync_copy(acc.at[pl.ds(sid * fl, fl)],
                        out_hbm.at[core, pl.ds(sid * fl, fl)])

    @pl.when(sid == 15)
    def _():
        pltpu.sync_copy(acc.at[pl.ds(15 * fl, N - 15 * fl)],
                        out_hbm.at[core, pl.ds(15 * fl, N - 15 * fl)])


_spmm_call = pl.kernel(
    _spmm_body,
    out_type=jax.ShapeDtypeStruct((2, N, HD), jnp.float32),
    mesh=plsc.VectorSubcoreMesh(core_axis_name="c", subcore_axis_name="s"),
    compiler_params=pltpu.CompilerParams(
        needs_layout_passes=False, use_tc_tiling_on_sc=False),
    scratch_types=[
        pltpu.VMEM_SHARED((ACC_ROWS, HD), jnp.float32),
        pltpu.VMEM((NK, CHUNK), jnp.int32),
        pltpu.VMEM((MEGA,), jnp.float32),
        pltpu.VMEM((CHUNK,), jnp.int32),
        pltpu.VMEM((CHUNK,), jnp.int32),
        pltpu.VMEM((CHUNK, HD), jnp.float32),
        pltpu.VMEM((CHUNK, HD), jnp.float32),
        pltpu.SemaphoreType.DMA,
        pltpu.SemaphoreType.DMA,
        pltpu.SemaphoreType.DMA,
        pltpu.SemaphoreType.DMA,
        pltpu.SemaphoreType.DMA,
    ],
)

_ZEROS = None


def _spmm(idx, val, xs):
    global _ZEROS
    if _ZEROS is None:
        _ZEROS = jnp.zeros((ZROWS, HD), jnp.float32)
    rows2 = idx[0].reshape(E // CHUNK, CHUNK)
    cols2 = idx[1].reshape(E // CHUNK, CHUNK)
    return _spmm_call(rows2, cols2, val, xs, _ZEROS)


def _split(x):
    return jnp.stack([x[:, :HD], x[:, HD:]])


_FEAT_BLK = 1000


def _feats_body(x_ref, w_ref, b_ref, o_ref):
    acc = jnp.dot(x_ref[...], w_ref[...], preferred_element_type=jnp.float32)
    acc = acc + b_ref[...]
    n = jnp.sqrt(jnp.sum(acc * acc, axis=1, keepdims=True))
    o_ref[...] = acc / jnp.maximum(n, 1e-12)


def _proj_l2(x, w, b):
    m, k = x.shape
    d = w.shape[1]
    grid = m // _FEAT_BLK
    return pl.pallas_call(
        _feats_body,
        grid=(grid,),
        in_specs=[
            pl.BlockSpec((_FEAT_BLK, k), lambda i: (i, 0)),
            pl.BlockSpec((k, d), lambda i: (0, 0)),
            pl.BlockSpec((1, d), lambda i: (0, 0)),
        ],
        out_specs=pl.BlockSpec((_FEAT_BLK, d), lambda i: (i, 0)),
        out_shape=jax.ShapeDtypeStruct((m, d), jnp.float32),
    )(x, w, b.reshape(1, d))


_FUSE_BLK = 1000


def _modal_body(ia, ib, iadj, ta, tb, tadj, wi, wt, o_ref):
    img = ia[...] + ib[...] + RIS_ADJ_LAMBDA * iadj[...]
    txt = ta[...] + tb[...] + RIS_ADJ_LAMBDA * tadj[...]
    o_ref[...] = wi[...] * img + wt[...] * txt


def _modal_combine(im1, im2, imadj, tx1, tx2, txadj, wi, wt):
    spec = pl.BlockSpec((2, _FUSE_BLK, HD), lambda i: (0, i, 0))
    wspec = pl.BlockSpec((_FUSE_BLK, 1), lambda i: (i, 0))
    return pl.pallas_call(
        _modal_body,
        grid=(N // _FUSE_BLK,),
        in_specs=[spec, spec, spec, spec, spec, spec, wspec, wspec],
        out_specs=spec,
        out_shape=jax.ShapeDtypeStruct((2, N, HD), jnp.float32),
    )(im1, im2, imadj, tx1, tx2, txadj, wi, wt)


def _final_body(m, g1, g2, o_ref):
    m0, m1 = m[0], m[1]
    s = jnp.sum(m0 * m0 + m1 * m1, axis=1, keepdims=True)
    inv = RIS_LAMBDA / jnp.maximum(jnp.sqrt(s), 1e-12)
    o_ref[:, :HD] = m0 + g1[0] + g2[0] + inv * m0
    o_ref[:, HD:] = m1 + g1[1] + g2[1] + inv * m1


def _final_combine(modal, g1, g2):
    spec = pl.BlockSpec((2, _FUSE_BLK, HD), lambda i: (0, i, 0))
    return pl.pallas_call(
        _final_body,
        grid=(N // _FUSE_BLK,),
        in_specs=[spec, spec, spec],
        out_specs=pl.BlockSpec((_FUSE_BLK, LATDIM), lambda i: (i, 0)),
        out_shape=jax.ShapeDtypeStruct((N, LATDIM), jnp.float32),
    )(modal, g1, g2)


def kernel(adj_idx, adj_val, image_adj_idx, image_adj_val, text_adj_idx,
           text_adj_val, att_image_list, att_text_list, uEmbeds, iEmbeds,
           image_embedding, text_embedding, Wi, bi, Wt, bt):
    image_feats_n = _proj_l2(image_embedding, Wi, bi)
    text_feats_n = _proj_l2(text_embedding, Wt, bt)

    u_s = _split(uEmbeds)
    i_s = _split(iEmbeds)
    ui_s = jnp.concatenate([u_s, i_s], axis=1)

    embedsImageAdj = _spmm(image_adj_idx, image_adj_val, ui_s)
    embedsTextAdj = _spmm(text_adj_idx, text_adj_val, ui_s)

    embedsImage1 = _spmm(adj_idx, adj_val,
                         jnp.concatenate([u_s, _split(image_feats_n)], axis=1))
    embedsImage2 = _spmm(adj_idx, adj_val,
                         jnp.concatenate([embedsImage1[:, :USER], i_s], axis=1))
    embedsText1 = _spmm(adj_idx, adj_val,
                        jnp.concatenate([u_s, _split(text_feats_n)], axis=1))
    embedsText2 = _spmm(adj_idx, adj_val,
                        jnp.concatenate([embedsText1[:, :USER], i_s], axis=1))

    weight_sum = att_image_list + att_text_list
    weight_sum = jnp.where(weight_sum == 0, jnp.ones_like(weight_sum), weight_sum)
    wi_att = (att_image_list / weight_sum)[:, None]
    wt_att = (att_text_list / weight_sum)[:, None]

    embedsModal = _modal_combine(embedsImage1, embedsImage2, embedsImageAdj,
                                 embedsText1, embedsText2, embedsTextAdj,
                                 wi_att, wt_att)

    g1 = _spmm(adj_idx, adj_val, embedsModal)
    g2 = _spmm(adj_idx, adj_val, g1)
    embeds = _final_combine(embedsModal, g1, g2)
    return (embeds[:USER], embeds[USER:])


# speedup vs baseline: 8.8173x; 1.5370x over previous
import jax
import jax.numpy as jnp
from jax import lax
from jax.experimental import pallas as pl
from jax.experimental.pallas import tpu as pltpu
from jax.experimental.pallas import tpu_sc as plsc

USER = 25000
ITEM = 25000
N = USER + ITEM
E = 800000
LATDIM = 64
HD = LATDIM // 2
RIS_ADJ_LAMBDA = 0.2
RIS_LAMBDA = 0.5

NSUB = 16
SPAN = E // NSUB
CHUNK = 80
MEGA = 10000
NK = MEGA // CHUNK
NM = SPAN // MEGA
ACC_ROWS = 50048
ZROWS = ACC_ROWS // NSUB


NBUF = 4
PRE = 2


def _spmm_body(rows_hbm, cols_hbm, vals_hbm, xs_hbm, zeros_hbm, out_hbm,
               acc, cols_v, vals_v, rows_a, rows_b, rows_c, rows_d,
               gath_a, gath_b, gath_c, gath_d,
               sem_s, sem_a, sem_b, sem_c, sem_d,
               sem_sa, sem_sb, sem_sc, sem_sd):
    core = lax.axis_index("c")
    sid = lax.axis_index("s")

    RB = (rows_a, rows_b, rows_c, rows_d)
    GB = (gath_a, gath_b, gath_c, gath_d)
    SG = (sem_a, sem_b, sem_c, sem_d)
    SS = (sem_sa, sem_sb, sem_sc, sem_sd)

    pltpu.sync_copy(zeros_hbm, acc.at[pl.ds(sid * ZROWS, ZROWS)])

    xsrc = xs_hbm.at[core]

    def start(cb, k, p):
        pltpu.async_copy(rows_hbm.at[cb + k], RB[p], SG[p])
        pltpu.async_copy(xsrc.at[cols_v.at[k]], GB[p], SG[p])

    def waitg(cb, k, p):
        pltpu.make_async_copy(rows_hbm.at[cb + k], RB[p], SG[p]).wait()
        pltpu.make_async_copy(xsrc.at[cols_v.at[k]], GB[p], SG[p]).wait()

    def compute(k, p):
        gath_buf = GB[p]
        off = k * CHUNK
        for j in range(CHUNK // 16):
            val16 = vals_v[pl.ds(off + j * 16, 16)]
            for i in range(16):
                e = j * 16 + i
                vv = lax.gather(
                    val16, jnp.full((16, 1), i, jnp.int32),
                    lax.GatherDimensionNumbers(
                        offset_dims=(), collapsed_slice_dims=(0,),
                        start_index_map=(0,)),
                    slice_sizes=(1,),
                    mode=lax.GatherScatterMode.PROMISE_IN_BOUNDS)
                for c in range(HD // 16):
                    sl = pl.ds(c * 16, 16)
                    gath_buf[e, sl] = gath_buf[e, sl] * vv

    def scat_start(p):
        pltpu.async_copy(GB[p], acc.at[RB[p]], SS[p], add=True)

    def scat_wait(p):
        pltpu.make_async_copy(GB[p], acc.at[RB[p]], SS[p]).wait()

    for p in range(NBUF):
        for e in range(CHUNK):
            for c in range(HD // 16):
                GB[p][e, pl.ds(c * 16, 16)] = jnp.zeros((16,), jnp.float32)
        for e in range(CHUNK // 16):
            RB[p][pl.ds(e * 16, 16)] = jnp.zeros((16,), jnp.int32)
    plsc.subcore_barrier()
    for p in range(NBUF):
        scat_start(p)

    def mega_body(m, _):
        base = sid * SPAN + m * MEGA
        cb = base // CHUNK
        d_cols = pltpu.async_copy(cols_hbm.at[pl.ds(cb, NK)], cols_v, sem_s)
        d_vals = pltpu.async_copy(vals_hbm.at[pl.ds(base, MEGA)], vals_v,
                                  sem_s)
        d_cols.wait()
        d_vals.wait()
        for k in range(PRE):
            scat_wait(k % NBUF)
            start(cb, k, k % NBUF)

        def quad_body(t, _):
            k0 = 4 * t
            for p in range(4):
                k = k0 + p
                waitg(cb, k, p)
                w = (p + PRE) % NBUF
                scat_wait(w)
                start(cb, k + PRE, w)
                compute(k, p)
                scat_start(p)
            return 0

        lax.fori_loop(0, (NK - 5) // 4, quad_body, 0)
        for k in range(NK - 5, NK):
            p = k % NBUF
            waitg(cb, k, p)
            if k + PRE < NK:
                w = (k + PRE) % NBUF
                scat_wait(w)
                start(cb, k + PRE, w)
            compute(k, p)
            scat_start(p)
        return 0

    lax.fori_loop(0, NM, mega_body, 0)
    for p in range(NBUF):
        scat_wait(p)
    plsc.subcore_barrier()

    fl = 3120

    @pl.when(sid < 15)
    def _():
        pltpu.sync_copy(acc.at[pl.ds(sid * fl, fl)],
                        out_hbm.at[core, pl.ds(sid * fl, fl)])

    @pl.when(sid == 15)
    def _():
        pltpu.sync_copy(acc.at[pl.ds(15 * fl, N - 15 * fl)],
                        out_hbm.at[core, pl.ds(15 * fl, N - 15 * fl)])


_spmm_call = pl.kernel(
    _spmm_body,
    out_type=jax.ShapeDtypeStruct((2, N, HD), jnp.float32),
    mesh=plsc.VectorSubcoreMesh(core_axis_name="c", subcore_axis_name="s"),
    compiler_params=pltpu.CompilerParams(
        needs_layout_passes=False, use_tc_tiling_on_sc=False),
    scratch_types=[
        pltpu.VMEM_SHARED((ACC_ROWS, HD), jnp.float32),
        pltpu.VMEM((NK, CHUNK), jnp.int32),
        pltpu.VMEM((MEGA,), jnp.float32),
        pltpu.VMEM((CHUNK,), jnp.int32),
        pltpu.VMEM((CHUNK,), jnp.int32),
        pltpu.VMEM((CHUNK,), jnp.int32),
        pltpu.VMEM((CHUNK,), jnp.int32),
        pltpu.VMEM((CHUNK, HD), jnp.float32),
        pltpu.VMEM((CHUNK, HD), jnp.float32),
        pltpu.VMEM((CHUNK, HD), jnp.float32),
        pltpu.VMEM((CHUNK, HD), jnp.float32),
    ] + [pltpu.SemaphoreType.DMA] * 9,
)

_ZEROS = None


def _spmm(idx, val, xs):
    global _ZEROS
    if _ZEROS is None:
        _ZEROS = jnp.zeros((ZROWS, HD), jnp.float32)
    rows2 = idx[0].reshape(E // CHUNK, CHUNK)
    cols2 = idx[1].reshape(E // CHUNK, CHUNK)
    return _spmm_call(rows2, cols2, val, xs, _ZEROS)


def _split(x):
    return jnp.stack([x[:, :HD], x[:, HD:]])


_FEAT_BLK = 1000


def _feats_body(x_ref, w_ref, b_ref, o_ref):
    acc = jnp.dot(x_ref[...], w_ref[...], preferred_element_type=jnp.float32)
    acc = acc + b_ref[...]
    n = jnp.sqrt(jnp.sum(acc * acc, axis=1, keepdims=True))
    o_ref[...] = acc / jnp.maximum(n, 1e-12)


def _proj_l2(x, w, b):
    m, k = x.shape
    d = w.shape[1]
    grid = m // _FEAT_BLK
    return pl.pallas_call(
        _feats_body,
        grid=(grid,),
        in_specs=[
            pl.BlockSpec((_FEAT_BLK, k), lambda i: (i, 0)),
            pl.BlockSpec((k, d), lambda i: (0, 0)),
            pl.BlockSpec((1, d), lambda i: (0, 0)),
        ],
        out_specs=pl.BlockSpec((_FEAT_BLK, d), lambda i: (i, 0)),
        out_shape=jax.ShapeDtypeStruct((m, d), jnp.float32),
    )(x, w, b.reshape(1, d))


_FUSE_BLK = 1000


def _modal_body(ia, ib, iadj, ta, tb, tadj, wi, wt, o_ref):
    img = ia[...] + ib[...] + RIS_ADJ_LAMBDA * iadj[...]
    txt = ta[...] + tb[...] + RIS_ADJ_LAMBDA * tadj[...]
    o_ref[...] = wi[...] * img + wt[...] * txt


def _modal_combine(im1, im2, imadj, tx1, tx2, txadj, wi, wt):
    spec = pl.BlockSpec((2, _FUSE_BLK, HD), lambda i: (0, i, 0))
    wspec = pl.BlockSpec((_FUSE_BLK, 1), lambda i: (i, 0))
    return pl.pallas_call(
        _modal_body,
        grid=(N // _FUSE_BLK,),
        in_specs=[spec, spec, spec, spec, spec, spec, wspec, wspec],
        out_specs=spec,
        out_shape=jax.ShapeDtypeStruct((2, N, HD), jnp.float32),
    )(im1, im2, imadj, tx1, tx2, txadj, wi, wt)


def _final_body(m, g1, g2, o_ref):
    m0, m1 = m[0], m[1]
    s = jnp.sum(m0 * m0 + m1 * m1, axis=1, keepdims=True)
    inv = RIS_LAMBDA / jnp.maximum(jnp.sqrt(s), 1e-12)
    o_ref[:, :HD] = m0 + g1[0] + g2[0] + inv * m0
    o_ref[:, HD:] = m1 + g1[1] + g2[1] + inv * m1


def _final_combine(modal, g1, g2):
    spec = pl.BlockSpec((2, _FUSE_BLK, HD), lambda i: (0, i, 0))
    return pl.pallas_call(
        _final_body,
        grid=(N // _FUSE_BLK,),
        in_specs=[spec, spec, spec],
        out_specs=pl.BlockSpec((_FUSE_BLK, LATDIM), lambda i: (i, 0)),
        out_shape=jax.ShapeDtypeStruct((N, LATDIM), jnp.float32),
    )(modal, g1, g2)


def kernel(adj_idx, adj_val, image_adj_idx, image_adj_val, text_adj_idx,
           text_adj_val, att_image_list, att_text_list, uEmbeds, iEmbeds,
           image_embedding, text_embedding, Wi, bi, Wt, bt):
    image_feats_n = _proj_l2(image_embedding, Wi, bi)
    text_feats_n = _proj_l2(text_embedding, Wt, bt)

    u_s = _split(uEmbeds)
    i_s = _split(iEmbeds)
    ui_s = jnp.concatenate([u_s, i_s], axis=1)

    embedsImageAdj = _spmm(image_adj_idx, image_adj_val, ui_s)
    embedsTextAdj = _spmm(text_adj_idx, text_adj_val, ui_s)

    embedsImage1 = _spmm(adj_idx, adj_val,
                         jnp.concatenate([u_s, _split(image_feats_n)], axis=1))
    embedsImage2 = _spmm(adj_idx, adj_val,
                         jnp.concatenate([embedsImage1[:, :USER], i_s], axis=1))
    embedsText1 = _spmm(adj_idx, adj_val,
                        jnp.concatenate([u_s, _split(text_feats_n)], axis=1))
    embedsText2 = _spmm(adj_idx, adj_val,
                        jnp.concatenate([embedsText1[:, :USER], i_s], axis=1))

    weight_sum = att_image_list + att_text_list
    weight_sum = jnp.where(weight_sum == 0, jnp.ones_like(weight_sum), weight_sum)
    wi_att = (att_image_list / weight_sum)[:, None]
    wt_att = (att_text_list / weight_sum)[:, None]

    embedsModal = _modal_combine(embedsImage1, embedsImage2, embedsImageAdj,
                                 embedsText1, embedsText2, embedsTextAdj,
                                 wi_att, wt_att)

    g1 = _spmm(adj_idx, adj_val, embedsModal)
    g2 = _spmm(adj_idx, adj_val, g1)
    embeds = _final_combine(embedsModal, g1, g2)
    return (embeds[:USER], embeds[USER:])
